# Initial kernel scaffold; baseline (speedup 1.0000x reference)
import jax
import jax.numpy as jnp
from jax.experimental import pallas as pl


def kernel(pointCloud):
    raise NotImplementedError("write your pallas kernel here")


# speedup vs baseline: 383.4680x; 383.4680x over previous
import jax
import jax.numpy as jnp
from jax.experimental import pallas as pl
from jax.experimental.pallas import tpu as pltpu

_PATCHES_PER_BATCH = 16
_K = 16
_W_NORMAL = 1.0
_W_SURFVAR = 1.0
_NEWTON_ITERS = 24


def _patch_body(xT_ref, xC_ref, out_ref):
    xt = xT_ref[...]
    xc = xC_ref[...]
    rx = xt[0, 0:1, :]
    ry = xt[0, 1:2, :]
    rz = xt[0, 2:3, :]
    cx = xc[0, 0]
    cy = xc[0, 1]
    cz = xc[0, 2]

    dx = cx - rx
    dy = cy - ry
    dz = cz - rz
    d2 = dx * dx + dy * dy + dz * dz

    work = d2
    for _ in range(_K):
        m = jnp.min(work, axis=0, keepdims=True)
        work = jnp.where(work == m, jnp.inf, work)
    sel = work != d2

    zero = jnp.zeros_like(dx)
    wdx = jnp.where(sel, dx, zero)
    wdy = jnp.where(sel, dy, zero)
    wdz = jnp.where(sel, dz, zero)

    def rsum(v):
        return jnp.sum(v, axis=0, keepdims=True)

    a11 = rsum(wdx * dx)
    a12 = rsum(wdx * dy)
    a13 = rsum(wdx * dz)
    a22 = rsum(wdy * dy)
    a23 = rsum(wdy * dz)
    a33 = rsum(wdz * dz)

    tr = a11 + a22 + a33
    inv = 1.0 / jnp.maximum(tr, 1e-30)
    b11 = a11 * inv
    b12 = a12 * inv
    b13 = a13 * inv
    b22 = a22 * inv
    b23 = a23 * inv
    b33 = a33 * inv

    ctr = b11 + b22 + b33
    c1 = (b11 * b22 - b12 * b12) + (b11 * b33 - b13 * b13) + (b22 * b33 - b23 * b23)
    c0 = (b11 * (b22 * b33 - b23 * b23)
          - b12 * (b12 * b33 - b23 * b13)
          + b13 * (b12 * b23 - b22 * b13))

    lam = jnp.zeros_like(c0)
    for _ in range(_NEWTON_ITERS):
        f = ((ctr - lam) * lam - c1) * lam + c0
        fp = (2.0 * ctr - 3.0 * lam) * lam - c1
        lam = lam - f / jnp.minimum(fp, -1e-30)
        lam = jnp.clip(lam, 0.0, 0.33334)
    sv = lam

    alpha = ctr - lam
    beta = c1 - lam * alpha
    s11 = b11 * b11 + b12 * b12 + b13 * b13
    s12 = b11 * b12 + b12 * b22 + b13 * b23
    s13 = b11 * b13 + b12 * b23 + b13 * b33
    s22 = b12 * b12 + b22 * b22 + b23 * b23
    s23 = b12 * b13 + b22 * b23 + b23 * b33
    s33 = b13 * b13 + b23 * b23 + b33 * b33
    m11 = s11 - alpha * b11 + beta
    m12 = s12 - alpha * b12
    m13 = s13 - alpha * b13
    m22 = s22 - alpha * b22 + beta
    m23 = s23 - alpha * b23
    m33 = s33 - alpha * b33 + beta

    n1 = m11 * m11 + m12 * m12 + m13 * m13
    n2 = m12 * m12 + m22 * m22 + m23 * m23
    n3 = m13 * m13 + m23 * m23 + m33 * m33
    use1 = (n1 >= n2) & (n1 >= n3)
    use2 = jnp.logical_not(use1) & (n2 >= n3)
    vx = jnp.where(use1, m11, jnp.where(use2, m12, m13))
    vy = jnp.where(use1, m12, jnp.where(use2, m22, m23))
    vz = jnp.where(use1, m13, jnp.where(use2, m23, m33))
    nn = vx * vx + vy * vy + vz * vz
    invn = jax.lax.rsqrt(jnp.maximum(nn, 1e-38))
    nx = jnp.abs(vx) * invn
    ny = jnp.abs(vy) * invn
    nz = jnp.abs(vz) * invn

    mm = jnp.float32(dx.shape[0])
    ss = (jnp.sum(nx * nx) - jnp.sum(nx) ** 2 / mm
          + jnp.sum(ny * ny) - jnp.sum(ny) ** 2 / mm
          + jnp.sum(nz * nz) - jnp.sum(nz) ** 2 / mm)
    svsum = jnp.sum(sv)

    lane = jax.lax.broadcasted_iota(jnp.int32, (1, 1, 128), 2)
    out_ref[...] = jnp.where(lane == 0, ss, jnp.where(lane == 1, svsum, 0.0))


def kernel(pointCloud):
    B, N, _ = pointCloud.shape
    P = B * _PATCHES_PER_BATCH
    M = N // _PATCHES_PER_BATCH
    x = pointCloud.reshape(P, M, 3)
    xT = jnp.transpose(x, (0, 2, 1))
    xC = xT[..., None]

    partials = pl.pallas_call(
        _patch_body,
        grid=(P,),
        in_specs=[
            pl.BlockSpec((1, 3, M), lambda p: (p, 0, 0)),
            pl.BlockSpec((1, 3, M, 1), lambda p: (p, 0, 0, 0)),
        ],
        out_specs=pl.BlockSpec((1, 1, 128), lambda p: (p, 0, 0)),
        out_shape=jax.ShapeDtypeStruct((P, 1, 128), jnp.float32),
        compiler_params=pltpu.CompilerParams(
            dimension_semantics=("arbitrary",),
        ),
    )(xT, xC)

    nss = jnp.sum(partials[:, 0, 0])
    svs = jnp.sum(partials[:, 0, 1])
    loss = nss / (P * M * 3) * _W_NORMAL + svs / (P * M) * _W_SURFVAR
    return loss.astype(jnp.float32)
